# baseline (speedup 1.0000x reference)
import functools
import math

import jax
import jax.numpy as jnp
from jax import lax
from jax.experimental import pallas as pl
from jax.experimental.pallas import tpu as pltpu
from jax.experimental.pallas import tpu_sc as plsc

VOCAB = 4634
VOCAB_PAD = 4736
EMBED_DIM = 64
SCALE = math.sqrt(EMBED_DIM)

_info = plsc.get_sparse_core_info()
_NC = _info.num_cores
_NS = _info.num_subcores
_NW = _NC * _NS
_KPW = EMBED_DIM // _NW


def _make_lookup(n_batch, n_seq, unroll=8):
    n_grp = n_batch // 16
    assert n_grp % unroll == 0 and n_seq % 2 == 0
    mesh = plsc.VectorSubcoreMesh(core_axis_name="c", subcore_axis_name="s")

    @functools.partial(
        pl.kernel,
        mesh=mesh,
        out_type=jax.ShapeDtypeStruct((n_seq, EMBED_DIM, n_batch), jnp.float32),
        scratch_types=[
            [pltpu.VMEM((VOCAB_PAD,), jnp.float32)] * _KPW,
            [pltpu.VMEM((n_batch,), jnp.int32)] * 2,
            [pltpu.VMEM((_KPW, n_batch), jnp.float32)] * 2,
            [pltpu.SemaphoreType.DMA] * 2,
            [pltpu.SemaphoreType.DMA] * 2,
            pltpu.SemaphoreType.DMA,
        ],
        compiler_params=pltpu.CompilerParams(needs_layout_passes=False),
    )
    def lookup_kernel(tabt_hbm, idxt_hbm, out_hbm, tk, idx_v, obuf, isem, osem, tsem):
        wid = lax.axis_index("s") * _NC + lax.axis_index("c")
        k0 = wid * _KPW

        for kk in range(_KPW):
            pltpu.async_copy(tabt_hbm.at[k0 + kk], tk[kk], tsem)
        for kk in range(_KPW):
            pltpu.make_async_copy(tabt_hbm.at[k0 + kk], tk[kk], tsem).wait()

        for b in range(2):
            pltpu.async_copy(
                idxt_hbm.at[pl.ds(b * n_batch, n_batch)], idx_v[b], isem[b]
            )

        def body(t, carry):
            for b in range(2):
                j = 2 * t + b
                pltpu.make_async_copy(
                    idxt_hbm.at[pl.ds(j * n_batch, n_batch)], idx_v[b], isem[b]
                ).wait()
                @pl.when(t >= 1)
                def _():
                    pltpu.make_async_copy(
                        obuf[b], out_hbm.at[j, pl.ds(k0, _KPW)], osem[b]
                    ).wait()

                @plsc.parallel_loop(0, n_grp, unroll=unroll)
                def _(g):
                    sl = pl.ds(g * 16, 16)
                    vidx = idx_v[b][sl]
                    for kk in range(_KPW):
                        obuf[b][kk, sl] = plsc.load_gather(tk[kk], [vidx]) * SCALE

                @pl.when(t < n_seq // 2 - 1)
                def _():
                    pltpu.async_copy(
                        idxt_hbm.at[pl.ds((j + 2) * n_batch, n_batch)],
                        idx_v[b],
                        isem[b],
                    )

                pltpu.async_copy(obuf[b], out_hbm.at[j, pl.ds(k0, _KPW)], osem[b])
            return carry

        lax.fori_loop(0, n_seq // 2, body, 0)

        for b in range(2):
            j = n_seq - 2 + b
            pltpu.make_async_copy(
                obuf[b], out_hbm.at[j, pl.ds(k0, _KPW)], osem[b]
            ).wait()

    return lookup_kernel


_lookup = _make_lookup(4096, 200, unroll=16)


def kernel(x, lut):
    tabt = jnp.zeros((EMBED_DIM, VOCAB_PAD), jnp.float32).at[:, :VOCAB].set(
        jnp.swapaxes(lut, 0, 1)
    )
    idxt = x.astype(jnp.int32).T.reshape(-1)
    out_t = _lookup(tabt, idxt)
    return jnp.transpose(out_t, (2, 0, 1))


# speedup vs baseline: 1.0940x; 1.0940x over previous
import functools
import math

import jax
import jax.numpy as jnp
from jax import lax
from jax.experimental import pallas as pl
from jax.experimental.pallas import tpu as pltpu
from jax.experimental.pallas import tpu_sc as plsc

VOCAB = 4634
VOCAB_PAD = 4736
EMBED_DIM = 64
SCALE = math.sqrt(EMBED_DIM)

_info = plsc.get_sparse_core_info()
_NC = _info.num_cores
_NS = _info.num_subcores
_NW = _NC * _NS
_KPW = EMBED_DIM // _NW


def _make_lookup(n_batch, n_seq, unroll=8):
    n_grp = n_batch // 16
    assert n_grp % unroll == 0 and n_seq % 2 == 0
    mesh = plsc.VectorSubcoreMesh(core_axis_name="c", subcore_axis_name="s")

    @functools.partial(
        pl.kernel,
        mesh=mesh,
        out_type=jax.ShapeDtypeStruct((n_seq, EMBED_DIM, n_batch), jnp.float32),
        scratch_types=[
            [pltpu.VMEM((VOCAB_PAD,), jnp.float32)] * _KPW,
            [pltpu.VMEM((n_batch,), jnp.int32)] * 2,
            [pltpu.VMEM((1, n_batch), jnp.float32)] * 2,
            [pltpu.SemaphoreType.DMA] * 2,
            [pltpu.SemaphoreType.DMA] * 2,
            pltpu.SemaphoreType.DMA,
        ],
        compiler_params=pltpu.CompilerParams(needs_layout_passes=False),
    )
    def lookup_kernel(tabt_hbm, idxt_hbm, out_hbm, tk, idx_v, obuf, isem, osem, tsem):
        wid = lax.axis_index("s") * _NC + lax.axis_index("c")
        k0 = wid * _KPW

        for kk in range(_KPW):
            pltpu.async_copy(tabt_hbm.at[k0 + kk], tk[kk], tsem)
        for kk in range(_KPW):
            pltpu.make_async_copy(tabt_hbm.at[k0 + kk], tk[kk], tsem).wait()

        for b in range(2):
            pltpu.async_copy(
                idxt_hbm.at[pl.ds(b * n_batch, n_batch)], idx_v[b], isem[b]
            )

        def body(t, carry):
            for b in range(2):
                j = 2 * t + b
                pltpu.make_async_copy(
                    idxt_hbm.at[pl.ds(j * n_batch, n_batch)], idx_v[b], isem[b]
                ).wait()
                @pl.when(t >= 1)
                def _():
                    pltpu.make_async_copy(
                        obuf[b], out_hbm.at[j, pl.ds(k0, 1)], osem[b]
                    ).wait()

                @plsc.parallel_loop(0, n_grp, unroll=unroll)
                def _(g):
                    sl = pl.ds(g * 16, 16)
                    vidx = idx_v[b][sl]
                    for kk in range(_KPW):
                        obuf[b][0, sl] = plsc.load_gather(tk[kk], [vidx]) * SCALE

                @pl.when(t < n_seq // 2 - 1)
                def _():
                    pltpu.async_copy(
                        idxt_hbm.at[pl.ds((j + 2) * n_batch, n_batch)],
                        idx_v[b],
                        isem[b],
                    )

                pltpu.async_copy(obuf[b], out_hbm.at[j, pl.ds(k0, 1)], osem[b])
            return carry

        lax.fori_loop(0, n_seq // 2, body, 0)

        for b in range(2):
            j = n_seq - 2 + b
            pltpu.make_async_copy(
                obuf[b], out_hbm.at[j, pl.ds(k0, 1)], osem[b]
            ).wait()

    return lookup_kernel


_lookup = _make_lookup(4096, 200, unroll=16)


def kernel(x, lut):
    tabt = jnp.zeros((EMBED_DIM, VOCAB_PAD), jnp.float32).at[:, :VOCAB].set(
        jnp.swapaxes(lut, 0, 1)
    )
    idxt = x.astype(jnp.int32).T.reshape(-1)
    out_t = _lookup(tabt, idxt)
    return jnp.transpose(out_t, (2, 0, 1))


# speedup vs baseline: 1.1338x; 1.0364x over previous
import functools
import math

import jax
import jax.numpy as jnp
from jax import lax
from jax.experimental import pallas as pl
from jax.experimental.pallas import tpu as pltpu
from jax.experimental.pallas import tpu_sc as plsc

VOCAB = 4634
VOCAB_PAD = 4736
EMBED_DIM = 64
SCALE = math.sqrt(EMBED_DIM)

_info = plsc.get_sparse_core_info()
_NC = _info.num_cores
_NS = _info.num_subcores
_NW = _NC * _NS
_KPW = EMBED_DIM // _NW


def _make_lookup(n_batch, n_seq, unroll=8, jb=4):
    n_grp = n_batch // 16
    assert (jb * n_grp) % unroll == 0 and n_seq % (2 * jb) == 0
    mesh = plsc.VectorSubcoreMesh(core_axis_name="c", subcore_axis_name="s")

    @functools.partial(
        pl.kernel,
        mesh=mesh,
        out_type=jax.ShapeDtypeStruct((n_seq, EMBED_DIM, n_batch), jnp.float32),
        scratch_types=[
            [pltpu.VMEM((VOCAB_PAD,), jnp.float32)] * _KPW,
            [pltpu.VMEM((jb * n_batch,), jnp.int32)] * 2,
            [pltpu.VMEM((jb, _KPW, n_batch), jnp.float32)] * 2,
            [pltpu.SemaphoreType.DMA] * 2,
            [pltpu.SemaphoreType.DMA] * 2,
            pltpu.SemaphoreType.DMA,
        ],
        compiler_params=pltpu.CompilerParams(needs_layout_passes=False),
    )
    def lookup_kernel(tabt_hbm, idxt_hbm, out_hbm, tk, idx_v, obuf, isem, osem, tsem):
        wid = lax.axis_index("s") * _NC + lax.axis_index("c")
        k0 = wid * _KPW

        for kk in range(_KPW):
            pltpu.async_copy(tabt_hbm.at[k0 + kk], tk[kk], tsem)
        for kk in range(_KPW):
            pltpu.make_async_copy(tabt_hbm.at[k0 + kk], tk[kk], tsem).wait()

        for b in range(2):
            pltpu.async_copy(
                idxt_hbm.at[pl.ds(b * jb * n_batch, jb * n_batch)], idx_v[b], isem[b]
            )

        def body(t, carry):
            for b in range(2):
                j0 = (2 * t + b) * jb
                pltpu.make_async_copy(
                    idxt_hbm.at[pl.ds(j0 * n_batch, jb * n_batch)], idx_v[b], isem[b]
                ).wait()
                @pl.when(t >= 1)
                def _():
                    pltpu.make_async_copy(
                        obuf[b], out_hbm.at[pl.ds(j0, jb), pl.ds(k0, _KPW)], osem[b]
                    ).wait()

                @plsc.parallel_loop(0, jb * n_grp, unroll=unroll)
                def _(g):
                    jj = g // n_grp
                    sl = pl.ds(g * 16, 16)
                    vidx = idx_v[b][sl]
                    for kk in range(_KPW):
                        vals = plsc.load_gather(tk[kk], [vidx]) * SCALE
                        obuf[b][jj, kk, pl.ds((g % n_grp) * 16, 16)] = vals

                @pl.when(t < n_seq // (2 * jb) - 1)
                def _():
                    pltpu.async_copy(
                        idxt_hbm.at[pl.ds((j0 + 2 * jb) * n_batch, jb * n_batch)],
                        idx_v[b],
                        isem[b],
                    )

                pltpu.async_copy(
                    obuf[b], out_hbm.at[pl.ds(j0, jb), pl.ds(k0, _KPW)], osem[b]
                )
            return carry

        lax.fori_loop(0, n_seq // (2 * jb), body, 0)

        for b in range(2):
            j0 = (n_seq - 2 * jb + b * jb)
            pltpu.make_async_copy(
                obuf[b], out_hbm.at[pl.ds(j0, jb), pl.ds(k0, _KPW)], osem[b]
            ).wait()

    return lookup_kernel


_lookup = _make_lookup(4096, 200, unroll=16, jb=4)


def kernel(x, lut):
    tabt = jnp.zeros((EMBED_DIM, VOCAB_PAD), jnp.float32).at[:, :VOCAB].set(
        jnp.swapaxes(lut, 0, 1)
    )
    idxt = x.astype(jnp.int32).T.reshape(-1)
    out_t = _lookup(tabt, idxt)
    return jnp.transpose(out_t, (2, 0, 1))


# speedup vs baseline: 1.7390x; 1.5338x over previous
import functools
import math

import jax
import jax.numpy as jnp
from jax import lax
from jax.experimental import pallas as pl
from jax.experimental.pallas import tpu as pltpu
from jax.experimental.pallas import tpu_sc as plsc

VOCAB = 4634
VOCAB_PAD = 4736
EMBED_DIM = 64
SCALE = math.sqrt(EMBED_DIM)

_info = plsc.get_sparse_core_info()
_NC = _info.num_cores
_NS = _info.num_subcores
_NW = _NC * _NS
_KPW = 8


def _make_lookup(n_batch, n_seq, unroll=8):
    n_grp = n_batch // 16
    assert n_grp % unroll == 0 and n_seq % 2 == 0
    mesh = plsc.VectorSubcoreMesh(core_axis_name="c", subcore_axis_name="s")

    @functools.partial(
        pl.kernel,
        mesh=mesh,
        out_type=jax.ShapeDtypeStruct((n_seq, EMBED_DIM, n_batch), jnp.float32),
        scratch_types=[
            [pltpu.VMEM((VOCAB_PAD,), jnp.float32)] * _KPW,
            [pltpu.VMEM((n_batch,), jnp.int32)] * 2,
            [pltpu.VMEM((_KPW, n_batch), jnp.float32)] * 2,
            [pltpu.SemaphoreType.DMA] * 2,
            [pltpu.SemaphoreType.DMA] * 2,
            pltpu.SemaphoreType.DMA,
        ],
        compiler_params=pltpu.CompilerParams(needs_layout_passes=False),
    )
    def lookup_kernel(tabt_hbm, idxt_hbm, out_hbm, tk, idx_v, obuf, isem, osem, tsem):
        wid = lax.axis_index("s") * _NC + lax.axis_index("c")
        n_slab = EMBED_DIM // _KPW
        jpg = n_seq // (_NW // n_slab)
        k0 = (wid % n_slab) * _KPW
        j0g = (wid // n_slab) * jpg

        for kk in range(_KPW):
            pltpu.async_copy(tabt_hbm.at[k0 + kk], tk[kk], tsem)
        for kk in range(_KPW):
            pltpu.make_async_copy(tabt_hbm.at[k0 + kk], tk[kk], tsem).wait()

        for b in range(2):
            pltpu.async_copy(
                idxt_hbm.at[pl.ds((j0g + b) * n_batch, n_batch)], idx_v[b], isem[b]
            )

        def body(t, carry):
            for b in range(2):
                j = j0g + 2 * t + b
                pltpu.make_async_copy(
                    idxt_hbm.at[pl.ds(j * n_batch, n_batch)], idx_v[b], isem[b]
                ).wait()
                @pl.when(t >= 1)
                def _():
                    pltpu.make_async_copy(
                        obuf[b], out_hbm.at[j, pl.ds(k0, _KPW)], osem[b]
                    ).wait()

                @plsc.parallel_loop(0, n_grp, unroll=unroll)
                def _(g):
                    sl = pl.ds(g * 16, 16)
                    vidx = idx_v[b][sl]
                    for kk in range(_KPW):
                        obuf[b][kk, sl] = plsc.load_gather(tk[kk], [vidx]) * SCALE

                @pl.when(t < jpg // 2 - 1)
                def _():
                    pltpu.async_copy(
                        idxt_hbm.at[pl.ds((j + 2) * n_batch, n_batch)],
                        idx_v[b],
                        isem[b],
                    )

                pltpu.async_copy(obuf[b], out_hbm.at[j, pl.ds(k0, _KPW)], osem[b])
            return carry

        lax.fori_loop(0, jpg // 2, body, 0)

        for b in range(2):
            j = j0g + jpg - 2 + b
            pltpu.make_async_copy(
                obuf[b], out_hbm.at[j, pl.ds(k0, _KPW)], osem[b]
            ).wait()

    return lookup_kernel


_lookup = _make_lookup(4096, 200, unroll=16)


def kernel(x, lut):
    tabt = jnp.zeros((EMBED_DIM, VOCAB_PAD), jnp.float32).at[:, :VOCAB].set(
        jnp.swapaxes(lut, 0, 1)
    )
    idxt = x.astype(jnp.int32).T.reshape(-1)
    out_t = _lookup(tabt, idxt)
    return jnp.transpose(out_t, (2, 0, 1))
